# baseline (speedup 1.0000x reference)
import functools

import jax
import jax.numpy as jnp
from jax import lax
from jax.experimental import pallas as pl
from jax.experimental.pallas import tpu as pltpu
from jax.experimental.pallas import tpu_sc as plsc

F32 = jnp.float32
I32 = jnp.int32
NC = 2
NS = 16
NW = NC * NS
LANES = 16

EMB = 64
HALF = EMB // 2


def _round_up(x, m):
    return (x + m - 1) // m * m


def _mesh():
    return plsc.VectorSubcoreMesh(core_axis_name="c", subcore_axis_name="s")


_SC_PARAMS = pltpu.CompilerParams(use_tc_tiling_on_sc=False, needs_layout_passes=False)


def _mlp2_tc(x, p, bn=400):
    n, d = x.shape
    w1, b1 = p["l1"]["W"], p["l1"]["b"]
    w2, b2 = p["l2"]["W"], p["l2"]["b"]
    h, o = w1.shape[1], w2.shape[1]

    def body(x_ref, w1_ref, b1_ref, w2_ref, b2_ref, o_ref):
        hid = jnp.maximum(
            jnp.dot(x_ref[...], w1_ref[...], preferred_element_type=F32)
            + b1_ref[...], 0.0)
        o_ref[...] = (jnp.dot(hid, w2_ref[...], preferred_element_type=F32)
                      + b2_ref[...])

    return pl.pallas_call(
        body,
        grid=(n // bn,),
        in_specs=[
            pl.BlockSpec((bn, d), lambda i: (i, 0)),
            pl.BlockSpec((d, h), lambda i: (0, 0)),
            pl.BlockSpec((1, h), lambda i: (0, 0)),
            pl.BlockSpec((h, o), lambda i: (0, 0)),
            pl.BlockSpec((1, o), lambda i: (0, 0)),
        ],
        out_specs=pl.BlockSpec((bn, o), lambda i: (i, 0)),
        out_shape=jax.ShapeDtypeStruct((n, o), F32),
    )(x, w1, b1.reshape(1, h), w2, b2.reshape(1, o))


def _msg_tc(x, p, bn=400):
    n = x.shape[0]
    w1, b1 = p["l1"]["W"], p["l1"]["b"]
    w2, b2 = p["l2"]["W"], p["l2"]["b"]

    def body(x_ref, w1_ref, b1_ref, w2_ref, b2_ref, lo_ref, hi_ref):
        hid = jnp.maximum(
            jnp.dot(x_ref[...], w1_ref[...], preferred_element_type=F32)
            + b1_ref[...], 0.0)
        out = (jnp.dot(hid, w2_ref[...], preferred_element_type=F32)
               + b2_ref[...])
        lo_ref[...] = out[:, :HALF]
        hi_ref[...] = out[:, HALF:]

    return pl.pallas_call(
        body,
        grid=(n // bn,),
        in_specs=[
            pl.BlockSpec((bn, EMB), lambda i: (i, 0)),
            pl.BlockSpec((EMB, EMB), lambda i: (0, 0)),
            pl.BlockSpec((1, EMB), lambda i: (0, 0)),
            pl.BlockSpec((EMB, EMB), lambda i: (0, 0)),
            pl.BlockSpec((1, EMB), lambda i: (0, 0)),
        ],
        out_specs=[
            pl.BlockSpec((bn, HALF), lambda i: (i, 0)),
            pl.BlockSpec((bn, HALF), lambda i: (i, 0)),
        ],
        out_shape=[
            jax.ShapeDtypeStruct((n, HALF), F32),
            jax.ShapeDtypeStruct((n, HALF), F32),
        ],
    )(x, w1, b1.reshape(1, EMB), w2, b2.reshape(1, EMB))


def _edgew_tc(hs, hd, ef, p, be=1000):
    e = hs.shape[0]
    de = ef.shape[1]
    w1, b1 = p["l1"]["W"], p["l1"]["b"]
    w2, b2 = p["l2"]["W"], p["l2"]["b"]
    ws, wd, we = w1[:EMB], w1[EMB:2 * EMB], w1[2 * EMB:]
    h = w1.shape[1]

    def body(hs_ref, hd_ref, ef_ref, ws_ref, wd_ref, we_ref, b1_ref,
             w2_ref, b2_ref, o_ref):
        hid = (jnp.dot(hs_ref[...], ws_ref[...], preferred_element_type=F32)
               + jnp.dot(hd_ref[...], wd_ref[...], preferred_element_type=F32)
               + jnp.dot(ef_ref[...], we_ref[...], preferred_element_type=F32)
               + b1_ref[...])
        hid = jnp.maximum(hid, 0.0)
        out = (jnp.dot(hid, w2_ref[...], preferred_element_type=F32)
               + b2_ref[...])
        o_ref[...] = jax.nn.sigmoid(out)

    return pl.pallas_call(
        body,
        grid=(e // be,),
        in_specs=[
            pl.BlockSpec((be, EMB), lambda i: (i, 0)),
            pl.BlockSpec((be, EMB), lambda i: (i, 0)),
            pl.BlockSpec((be, de), lambda i: (i, 0)),
            pl.BlockSpec((EMB, h), lambda i: (0, 0)),
            pl.BlockSpec((EMB, h), lambda i: (0, 0)),
            pl.BlockSpec((de, h), lambda i: (0, 0)),
            pl.BlockSpec((1, h), lambda i: (0, 0)),
            pl.BlockSpec((h, 1), lambda i: (0, 0)),
            pl.BlockSpec((1, 1), lambda i: (0, 0)),
        ],
        out_specs=pl.BlockSpec((be, 1), lambda i: (i, 0)),
        out_shape=jax.ShapeDtypeStruct((e, 1), F32),
    )(hs, hd, ef, ws, wd, we, b1.reshape(1, h), w2, b2.reshape(1, 1))


def _update_tc(agg_lo, agg_hi, deg, node, p, bn=400):
    n = node.shape[0]
    wg, bg = p["gate"]["W"], p["gate"]["b"]
    wu1, bu1 = p["update"]["l1"]["W"], p["update"]["l1"]["b"]
    wu2, bu2 = p["update"]["l2"]["W"], p["update"]["l2"]["b"]
    lng, lnb = p["ln_g"], p["ln_b"]

    def body(lo_ref, hi_ref, dg_ref, nd_ref,
             wg_lo, wg_hi, wg_nd, bg_ref,
             wu_lo, wu_hi, wu_nd, bu1_ref, wu2_ref, bu2_ref,
             g_ref, b_ref, o_ref):
        d = jnp.maximum(dg_ref[...], 1.0)
        lo = lo_ref[...] / d
        hi = hi_ref[...] / d
        nd = nd_ref[...]
        gate = jax.nn.sigmoid(
            jnp.dot(lo, wg_lo[...], preferred_element_type=F32)
            + jnp.dot(hi, wg_hi[...], preferred_element_type=F32)
            + jnp.dot(nd, wg_nd[...], preferred_element_type=F32)
            + bg_ref[...])
        hid = jnp.maximum(
            jnp.dot(lo, wu_lo[...], preferred_element_type=F32)
            + jnp.dot(hi, wu_hi[...], preferred_element_type=F32)
            + jnp.dot(nd, wu_nd[...], preferred_element_type=F32)
            + bu1_ref[...], 0.0)
        upd = (jnp.dot(hid, wu2_ref[...], preferred_element_type=F32)
               + bu2_ref[...])
        out = gate * upd + (1.0 - gate) * nd
        m = jnp.mean(out, axis=-1, keepdims=True)
        v = jnp.mean((out - m) ** 2, axis=-1, keepdims=True)
        o_ref[...] = (g_ref[...] * (out - m) / jnp.sqrt(v + 1e-3)
                      + b_ref[...])

    full = lambda shp: pl.BlockSpec(shp, lambda i: (0, 0))
    return pl.pallas_call(
        body,
        grid=(n // bn,),
        in_specs=[
            pl.BlockSpec((bn, HALF), lambda i: (i, 0)),
            pl.BlockSpec((bn, HALF), lambda i: (i, 0)),
            pl.BlockSpec((bn, 1), lambda i: (i, 0)),
            pl.BlockSpec((bn, EMB), lambda i: (i, 0)),
            full((HALF, EMB)), full((HALF, EMB)), full((EMB, EMB)),
            full((1, EMB)),
            full((HALF, EMB)), full((HALF, EMB)), full((EMB, EMB)),
            full((1, EMB)), full((EMB, EMB)), full((1, EMB)),
            full((1, EMB)), full((1, EMB)),
        ],
        out_specs=pl.BlockSpec((bn, EMB), lambda i: (i, 0)),
        out_shape=jax.ShapeDtypeStruct((n, EMB), F32),
    )(agg_lo, agg_hi, deg, node,
      wg[:HALF], wg[HALF:EMB], wg[EMB:], bg.reshape(1, EMB),
      wu1[:HALF], wu1[HALF:EMB], wu1[EMB:], bu1.reshape(1, EMB),
      wu2, bu2.reshape(1, EMB),
      lng.reshape(1, EMB), lnb.reshape(1, EMB))


@functools.lru_cache(None)
def _pair_gather_sc(e_vc, e_vk):
    b = 1000
    assert (e_vc // NW) % b == 0 and (e_vk // NW) % b == 0

    @functools.partial(
        pl.kernel,
        mesh=_mesh(),
        compiler_params=_SC_PARAMS,
        out_type=[jax.ShapeDtypeStruct((e_vc, EMB), F32),
                  jax.ShapeDtypeStruct((e_vc, EMB), F32),
                  jax.ShapeDtypeStruct((e_vk, EMB), F32),
                  jax.ShapeDtypeStruct((e_vk, EMB), F32)],
        scratch_types=[pltpu.VMEM((b,), I32),
                       pltpu.VMEM((b, EMB), F32),
                       pltpu.SemaphoreType.DMA],
    )
    def k(tvar, tcons, tcut, vcs_hbm, vcd_hbm, vks_hbm, vkd_hbm,
          ovcs, ovcd, ovks, ovkd, idx_v, rows_v, sem):
        wid = lax.axis_index("s") * NC + lax.axis_index("c")

        def phase(tsrc, tdst, src_hbm, dst_hbm, os_hbm, od_hbm, e):
            epw = e // NW
            base0 = wid * epw

            def bdy(i, carry):
                base = base0 + i * b
                pltpu.sync_copy(src_hbm.at[pl.ds(base, b)], idx_v)
                pltpu.async_copy(tsrc.at[idx_v], rows_v, sem).wait()
                pltpu.sync_copy(rows_v, os_hbm.at[pl.ds(base, b)])
                pltpu.sync_copy(dst_hbm.at[pl.ds(base, b)], idx_v)
                pltpu.async_copy(tdst.at[idx_v], rows_v, sem).wait()
                pltpu.sync_copy(rows_v, od_hbm.at[pl.ds(base, b)])
                return carry

            lax.fori_loop(0, epw // b, bdy, 0)

        phase(tvar, tcons, vcs_hbm, vcd_hbm, ovcs, ovcd, e_vc)
        phase(tvar, tcut, vks_hbm, vkd_hbm, ovks, ovkd, e_vk)

    return k


@functools.lru_cache(None)
def _deg_sc(nvar, ncons, ncut, e_vc, e_vk):
    b = 1000
    g = (b + LANES - 1) // LANES
    buf = g * LANES
    np50 = _round_up(ncons, 256)
    np10 = _round_up(ncut, 256)
    assert nvar == ncons

    @functools.partial(
        pl.kernel,
        mesh=_mesh(),
        compiler_params=_SC_PARAMS,
        out_type=[jax.ShapeDtypeStruct((3, NC, np50), F32),
                  jax.ShapeDtypeStruct((NC, np10), F32)],
        scratch_types=[pltpu.VMEM((buf,), I32),
                       pltpu.VMEM((buf,), F32),
                       pltpu.VMEM((np50,), F32),
                       pltpu.VMEM((np50 // NS,), F32),
                       pltpu.VMEM((np50 // NS,), F32),
                       pltpu.VMEM_SHARED((NS, np50), F32)],
    )
    def k(vcd_hbm, vcs_hbm, vcw_hbm, vkd_hbm, vks_hbm, vkw_hbm,
          out50, out10, dst_v, w_v, acc_v, red_v, tmp_v, shr):
        c = lax.axis_index("c")
        s = lax.axis_index("s")
        wid = s * NC + c
        zf = jnp.zeros((LANES,), F32)
        zi = jnp.zeros((LANES,), I32)
        lane = lax.iota(I32, LANES)

        dst_v[pl.ds(buf - LANES, LANES)] = zi
        w_v[pl.ds(buf - LANES, LANES)] = zf

        def phase(dst_hbm, w_hbm, e, np_, write_out, first):
            epw = e // NW
            span = np_ // NS

            def z(i, carry):
                acc_v[pl.ds(i * LANES, LANES)] = zf
                return carry

            lax.fori_loop(0, np_ // LANES, z, 0)
            base0 = wid * epw

            def bdy(i, carry):
                base = base0 + i * b
                pltpu.sync_copy(dst_hbm.at[pl.ds(base, b)],
                                dst_v.at[pl.ds(0, b)])
                pltpu.sync_copy(w_hbm.at[pl.ds(base, b)],
                                w_v.at[pl.ds(0, b)])

                def grp(j, cc):
                    msk = (j * LANES + lane) < b
                    d16 = dst_v[pl.ds(j * LANES, LANES)]
                    w16 = w_v[pl.ds(j * LANES, LANES)]
                    plsc.addupdate_scatter(acc_v, [d16], w16, mask=msk)
                    return cc

                lax.fori_loop(0, g, grp, 0)
                return carry

            lax.fori_loop(0, epw // b, bdy, 0)

            if not first:
                plsc.subcore_barrier()
            pltpu.sync_copy(acc_v.at[pl.ds(0, np_)], shr.at[s, pl.ds(0, np_)])
            plsc.subcore_barrier()
            pltpu.sync_copy(shr.at[0, pl.ds(s * span, span)],
                            red_v.at[pl.ds(0, span)])

            def red(i, carry):
                pltpu.sync_copy(shr.at[i, pl.ds(s * span, span)],
                                tmp_v.at[pl.ds(0, span)])

                def add(j, cc):
                    red_v[pl.ds(j * LANES, LANES)] = (
                        red_v[pl.ds(j * LANES, LANES)]
                        + tmp_v[pl.ds(j * LANES, LANES)])
                    return cc

                lax.fori_loop(0, span // LANES, add, 0)
                return carry

            lax.fori_loop(1, NS, red, 0)
            write_out(span)

        phase(vcd_hbm, vcw_hbm, e_vc, np50,
              lambda sp: pltpu.sync_copy(
                  red_v, out50.at[0, c, pl.ds(s * sp, sp)]), True)
        phase(vcs_hbm, vcw_hbm, e_vc, np50,
              lambda sp: pltpu.sync_copy(
                  red_v, out50.at[1, c, pl.ds(s * sp, sp)]), False)
        phase(vks_hbm, vkw_hbm, e_vk, np50,
              lambda sp: pltpu.sync_copy(
                  red_v, out50.at[2, c, pl.ds(s * sp, sp)]), False)
        phase(vkd_hbm, vkw_hbm, e_vk, np10,
              lambda sp: pltpu.sync_copy(
                  red_v.at[pl.ds(0, sp)],
                  out10.at[c, pl.ds(s * sp, sp)]), False)

    return k


@functools.lru_cache(None)
def _agg_sc(nsrc, ndst, e):
    eps = e // NS
    b = 400
    bp = b
    nb = eps // b
    nb2 = nb // 2
    odd = nb % 2 == 1
    assert eps % b == 0 and bp % LANES == 0
    nch_n = 1
    while _round_up(-(-ndst // nch_n), 256) + 256 > 51000:
        nch_n += 1
    nch = _round_up(-(-ndst // nch_n), 256)
    np_ = nch * nch_n
    acc_rows = nch + 256
    span = nch // NS
    wch = span
    while wch > b:
        wch //= 2
    assert span % wch == 0 and wch % 8 == 0
    nwr = span // wch
    g16 = bp // LANES

    @functools.partial(
        pl.kernel,
        mesh=_mesh(),
        compiler_params=_SC_PARAMS,
        out_type=jax.ShapeDtypeStruct((NC, np_, HALF), F32),
        scratch_types=[pltpu.VMEM((bp,), I32), pltpu.VMEM((bp,), I32),
                       pltpu.VMEM((bp,), I32), pltpu.VMEM((bp,), I32),
                       pltpu.VMEM((bp,), F32), pltpu.VMEM((bp,), F32),
                       pltpu.VMEM((bp, HALF), F32),
                       pltpu.VMEM((bp, HALF), F32),
                       pltpu.VMEM_SHARED((acc_rows, HALF), F32),
                       pltpu.SemaphoreType.DMA,
                       pltpu.SemaphoreType.DMA],
    )
    def k(mlo, mhi, src_hbm, dst_hbm, w_hbm, out_hbm,
          src0, src1, dst0, dst1, w0, w1, rows0, rows1, acc, sem0, sem1):
        c = lax.axis_index("c")
        s = lax.axis_index("s")
        zf = jnp.zeros((LANES,), F32)
        srcs, dsts, ws = (src0, src1), (dst0, dst1), (w0, w1)
        rows, sems = (rows0, rows1), (sem0, sem1)

        def load_idx(i, t):
            base = s * eps + i * b
            pltpu.sync_copy(src_hbm.at[pl.ds(base, b)], srcs[t])
            pltpu.sync_copy(dst_hbm.at[pl.ds(base, b)], dsts[t])
            pltpu.sync_copy(w_hbm.at[pl.ds(base, b)], ws[t])

        def process(t, lo):
            pltpu.make_async_copy(mlo.at[pl.ds(0, bp)], rows[t],
                                  sems[t]).wait()
            if nch_n > 1:
                def rb(j, cc):
                    d16 = dsts[t][pl.ds(j * LANES, LANES)]
                    loc = d16 - lo
                    ok = (d16 >= lo) & (d16 < lo + nch)
                    dump = jnp.full((LANES,), nch, I32)
                    dsts[t][pl.ds(j * LANES, LANES)] = (
                        jnp.where(ok, loc, dump))
                    return cc

                lax.fori_loop(0, g16, rb, 0)

            def sc8(e0, cc):
                for u in range(8):
                    ei = e0 * 8 + u
                    wv = plsc.load_gather(
                        ws[t], [jnp.full((LANES,), 0, I32) + ei])
                    rows[t][ei, pl.ds(0, LANES)] = (
                        rows[t][ei, pl.ds(0, LANES)] * wv)
                    rows[t][ei, pl.ds(LANES, LANES)] = (
                        rows[t][ei, pl.ds(LANES, LANES)] * wv)
                return cc

            lax.fori_loop(0, bp // 8, sc8, 0)
            pltpu.sync_copy(rows[t], acc.at[dsts[t]], add=True)

        for h in range(nch_n):
            lo = h * nch

            def z(i, carry):
                rows0[i, pl.ds(0, LANES)] = zf
                rows0[i, pl.ds(LANES, LANES)] = zf
                return carry

            lax.fori_loop(0, wch, z, 0)
            for i in range(nwr):
                pltpu.sync_copy(rows0.at[pl.ds(0, wch)],
                                acc.at[pl.ds(s * span + i * wch, wch)])
            plsc.subcore_barrier()

            def run(tbl):
                load_idx(0, 0)
                pltpu.async_copy(tbl.at[src0], rows0, sem0)

                def bdy(i2, carry):
                    load_idx(i2 * 2 + 1, 1)
                    pltpu.async_copy(tbl.at[src1], rows1, sem1)
                    process(0, lo)

                    if odd:
                        load_idx(i2 * 2 + 2, 0)
                        pltpu.async_copy(tbl.at[src0], rows0, sem0)
                    else:
                        @pl.when(i2 < nb2 - 1)
                        def _():
                            load_idx(i2 * 2 + 2, 0)
                            pltpu.async_copy(tbl.at[src0], rows0, sem0)

                    process(1, lo)
                    return carry

                lax.fori_loop(0, nb2, bdy, 0)
                if odd:
                    process(0, lo)

            @pl.when(c == 0)
            def _():
                run(mlo)

            @pl.when(c == 1)
            def _():
                run(mhi)

            plsc.subcore_barrier()
            for i in range(nwr):
                off = s * span + i * wch
                pltpu.sync_copy(acc.at[pl.ds(off, wch)],
                                rows0.at[pl.ds(0, wch)])
                pltpu.sync_copy(rows0.at[pl.ds(0, wch)],
                                out_hbm.at[c, pl.ds(lo + off, wch)])
            if h + 1 < nch_n:
                plsc.subcore_barrier()

    return k


def _message_pass(node, neigh, src, dst, w, deg, p):
    n = node.shape[0]
    mlo, mhi = _msg_tc(neigh, p["message"])
    agg2 = _agg_sc(neigh.shape[0], n, src.shape[0])(mlo, mhi, src, dst, w)
    agg_lo = agg2[0, :n, :]
    agg_hi = agg2[1, :n, :]
    return _update_tc(agg_lo, agg_hi, deg, node, p)


def kernel(variable_features, constraint_features, cut_features,
           var_cons_edge_features, var_cut_edge_features,
           var_cons_edges, var_cut_edges, params):
    n_var = variable_features.shape[0]
    n_cons = constraint_features.shape[0]
    n_cut = cut_features.shape[0]
    e_vc = var_cons_edges.shape[1]
    e_vk = var_cut_edges.shape[1]

    h_var = _mlp2_tc(variable_features, params["var_emb"])
    h_cons = _mlp2_tc(constraint_features, params["cons_emb"])
    h_cut = _mlp2_tc(cut_features, params["cut_emb"])

    vc_s = var_cons_edges[0]
    vc_d = var_cons_edges[1]
    vk_s = var_cut_edges[0]
    vk_d = var_cut_edges[1]

    gvcs, gvcd, gvks, gvkd = _pair_gather_sc(e_vc, e_vk)(
        h_var, h_cons, h_cut, vc_s, vc_d, vk_s, vk_d)
    vc_w = _edgew_tc(gvcs, gvcd, var_cons_edge_features, params["ew_vc"])
    vk_w = _edgew_tc(gvks, gvkd, var_cut_edge_features, params["ew_vk"])

    vc_w1 = vc_w.reshape(e_vc)
    vk_w1 = vk_w.reshape(e_vk)

    d50, d10 = _deg_sc(n_var, n_cons, n_cut, e_vc, e_vk)(
        vc_d, vc_s, vc_w1, vk_d, vk_s, vk_w1)
    deg_v2c = (d50[0, 0, :n_cons] + d50[0, 1, :n_cons]).reshape(n_cons, 1)
    deg_c2v = (d50[1, 0, :n_var] + d50[1, 1, :n_var]).reshape(n_var, 1)
    deg_k2v = (d50[2, 0, :n_var] + d50[2, 1, :n_var]).reshape(n_var, 1)
    deg_v2k = (d10[0, :n_cut] + d10[1, :n_cut]).reshape(n_cut, 1)

    for l in range(4):
        h_cons = _message_pass(h_cons, h_var, vc_s, vc_d, vc_w1, deg_v2c,
                               params["mp_v2c"][l])
        h_var = _message_pass(h_var, h_cons, vc_d, vc_s, vc_w1, deg_c2v,
                              params["mp_c2v"][l])
        h_cut = _message_pass(h_cut, h_var, vk_s, vk_d, vk_w1, deg_v2k,
                              params["mp_v2k"][l])
        h_var = _message_pass(h_var, h_cut, vk_d, vk_s, vk_w1, deg_k2v,
                              params["mp_k2v"][l])
    return h_cut


# speedup vs baseline: 1.0350x; 1.0350x over previous
import functools

import jax
import jax.numpy as jnp
from jax import lax
from jax.experimental import pallas as pl
from jax.experimental.pallas import tpu as pltpu
from jax.experimental.pallas import tpu_sc as plsc

F32 = jnp.float32
I32 = jnp.int32
NC = 2
NS = 16
NW = NC * NS
LANES = 16

EMB = 64
HALF = EMB // 2


def _round_up(x, m):
    return (x + m - 1) // m * m


def _mesh():
    return plsc.VectorSubcoreMesh(core_axis_name="c", subcore_axis_name="s")


_SC_PARAMS = pltpu.CompilerParams(use_tc_tiling_on_sc=False, needs_layout_passes=False)


def _mlp2_tc(x, p, bn=400):
    n, d = x.shape
    w1, b1 = p["l1"]["W"], p["l1"]["b"]
    w2, b2 = p["l2"]["W"], p["l2"]["b"]
    h, o = w1.shape[1], w2.shape[1]

    def body(x_ref, w1_ref, b1_ref, w2_ref, b2_ref, o_ref):
        hid = jnp.maximum(
            jnp.dot(x_ref[...], w1_ref[...], preferred_element_type=F32)
            + b1_ref[...], 0.0)
        o_ref[...] = (jnp.dot(hid, w2_ref[...], preferred_element_type=F32)
                      + b2_ref[...])

    return pl.pallas_call(
        body,
        grid=(n // bn,),
        in_specs=[
            pl.BlockSpec((bn, d), lambda i: (i, 0)),
            pl.BlockSpec((d, h), lambda i: (0, 0)),
            pl.BlockSpec((1, h), lambda i: (0, 0)),
            pl.BlockSpec((h, o), lambda i: (0, 0)),
            pl.BlockSpec((1, o), lambda i: (0, 0)),
        ],
        out_specs=pl.BlockSpec((bn, o), lambda i: (i, 0)),
        out_shape=jax.ShapeDtypeStruct((n, o), F32),
    )(x, w1, b1.reshape(1, h), w2, b2.reshape(1, o))


def _msg_tc(x, p, bn=400):
    n = x.shape[0]
    w1, b1 = p["l1"]["W"], p["l1"]["b"]
    w2, b2 = p["l2"]["W"], p["l2"]["b"]

    def body(x_ref, w1_ref, b1_ref, w2_ref, b2_ref, o_ref):
        hid = jnp.maximum(
            jnp.dot(x_ref[...], w1_ref[...], preferred_element_type=F32)
            + b1_ref[...], 0.0)
        out = (jnp.dot(hid, w2_ref[...], preferred_element_type=F32)
               + b2_ref[...])
        o_ref[0] = out[:, :HALF]
        o_ref[1] = out[:, HALF:]

    return pl.pallas_call(
        body,
        grid=(n // bn,),
        in_specs=[
            pl.BlockSpec((bn, EMB), lambda i: (i, 0)),
            pl.BlockSpec((EMB, EMB), lambda i: (0, 0)),
            pl.BlockSpec((1, EMB), lambda i: (0, 0)),
            pl.BlockSpec((EMB, EMB), lambda i: (0, 0)),
            pl.BlockSpec((1, EMB), lambda i: (0, 0)),
        ],
        out_specs=pl.BlockSpec((2, bn, HALF), lambda i: (0, i, 0)),
        out_shape=jax.ShapeDtypeStruct((2, n, HALF), F32),
    )(x, w1, b1.reshape(1, EMB), w2, b2.reshape(1, EMB))


def _edgew_tc(hs, hd, ef, p, be=1000):
    e = hs.shape[0]
    de = ef.shape[1]
    w1, b1 = p["l1"]["W"], p["l1"]["b"]
    w2, b2 = p["l2"]["W"], p["l2"]["b"]
    ws, wd, we = w1[:EMB], w1[EMB:2 * EMB], w1[2 * EMB:]
    h = w1.shape[1]

    def body(hs_ref, hd_ref, ef_ref, ws_ref, wd_ref, we_ref, b1_ref,
             w2_ref, b2_ref, o_ref):
        hid = (jnp.dot(hs_ref[...], ws_ref[...], preferred_element_type=F32)
               + jnp.dot(hd_ref[...], wd_ref[...], preferred_element_type=F32)
               + jnp.dot(ef_ref[...], we_ref[...], preferred_element_type=F32)
               + b1_ref[...])
        hid = jnp.maximum(hid, 0.0)
        out = (jnp.dot(hid, w2_ref[...], preferred_element_type=F32)
               + b2_ref[...])
        o_ref[...] = jax.nn.sigmoid(out)

    return pl.pallas_call(
        body,
        grid=(e // be,),
        in_specs=[
            pl.BlockSpec((be, EMB), lambda i: (i, 0)),
            pl.BlockSpec((be, EMB), lambda i: (i, 0)),
            pl.BlockSpec((be, de), lambda i: (i, 0)),
            pl.BlockSpec((EMB, h), lambda i: (0, 0)),
            pl.BlockSpec((EMB, h), lambda i: (0, 0)),
            pl.BlockSpec((de, h), lambda i: (0, 0)),
            pl.BlockSpec((1, h), lambda i: (0, 0)),
            pl.BlockSpec((h, 1), lambda i: (0, 0)),
            pl.BlockSpec((1, 1), lambda i: (0, 0)),
        ],
        out_specs=pl.BlockSpec((be, 1), lambda i: (i, 0)),
        out_shape=jax.ShapeDtypeStruct((e, 1), F32),
    )(hs, hd, ef, ws, wd, we, b1.reshape(1, h), w2, b2.reshape(1, 1))


def _update_tc(agg_lo, agg_hi, deg, node, p, bn=400):
    n = node.shape[0]
    wg, bg = p["gate"]["W"], p["gate"]["b"]
    wu1, bu1 = p["update"]["l1"]["W"], p["update"]["l1"]["b"]
    wu2, bu2 = p["update"]["l2"]["W"], p["update"]["l2"]["b"]
    lng, lnb = p["ln_g"], p["ln_b"]

    def body(lo_ref, hi_ref, dg_ref, nd_ref,
             wg_lo, wg_hi, wg_nd, bg_ref,
             wu_lo, wu_hi, wu_nd, bu1_ref, wu2_ref, bu2_ref,
             g_ref, b_ref, o_ref):
        d = jnp.maximum(dg_ref[...], 1.0)
        lo = lo_ref[...] / d
        hi = hi_ref[...] / d
        nd = nd_ref[...]
        gate = jax.nn.sigmoid(
            jnp.dot(lo, wg_lo[...], preferred_element_type=F32)
            + jnp.dot(hi, wg_hi[...], preferred_element_type=F32)
            + jnp.dot(nd, wg_nd[...], preferred_element_type=F32)
            + bg_ref[...])
        hid = jnp.maximum(
            jnp.dot(lo, wu_lo[...], preferred_element_type=F32)
            + jnp.dot(hi, wu_hi[...], preferred_element_type=F32)
            + jnp.dot(nd, wu_nd[...], preferred_element_type=F32)
            + bu1_ref[...], 0.0)
        upd = (jnp.dot(hid, wu2_ref[...], preferred_element_type=F32)
               + bu2_ref[...])
        out = gate * upd + (1.0 - gate) * nd
        m = jnp.mean(out, axis=-1, keepdims=True)
        v = jnp.mean((out - m) ** 2, axis=-1, keepdims=True)
        o_ref[...] = (g_ref[...] * (out - m) / jnp.sqrt(v + 1e-3)
                      + b_ref[...])

    full = lambda shp: pl.BlockSpec(shp, lambda i: (0, 0))
    return pl.pallas_call(
        body,
        grid=(n // bn,),
        in_specs=[
            pl.BlockSpec((bn, HALF), lambda i: (i, 0)),
            pl.BlockSpec((bn, HALF), lambda i: (i, 0)),
            pl.BlockSpec((bn, 1), lambda i: (i, 0)),
            pl.BlockSpec((bn, EMB), lambda i: (i, 0)),
            full((HALF, EMB)), full((HALF, EMB)), full((EMB, EMB)),
            full((1, EMB)),
            full((HALF, EMB)), full((HALF, EMB)), full((EMB, EMB)),
            full((1, EMB)), full((EMB, EMB)), full((1, EMB)),
            full((1, EMB)), full((1, EMB)),
        ],
        out_specs=pl.BlockSpec((bn, EMB), lambda i: (i, 0)),
        out_shape=jax.ShapeDtypeStruct((n, EMB), F32),
    )(agg_lo, agg_hi, deg, node,
      wg[:HALF], wg[HALF:EMB], wg[EMB:], bg.reshape(1, EMB),
      wu1[:HALF], wu1[HALF:EMB], wu1[EMB:], bu1.reshape(1, EMB),
      wu2, bu2.reshape(1, EMB),
      lng.reshape(1, EMB), lnb.reshape(1, EMB))


@functools.lru_cache(None)
def _pair_gather_sc(nsrc, ndst, e):
    epw = e // NW
    b = 1000
    nb = epw // b
    assert epw % b == 0

    @functools.partial(
        pl.kernel,
        mesh=_mesh(),
        compiler_params=_SC_PARAMS,
        out_type=[jax.ShapeDtypeStruct((e, EMB), F32),
                  jax.ShapeDtypeStruct((e, EMB), F32)],
        scratch_types=[pltpu.VMEM((b,), I32),
                       pltpu.VMEM((b, EMB), F32),
                       pltpu.SemaphoreType.DMA],
    )
    def k(tsrc, tdst, src_hbm, dst_hbm, os_hbm, od_hbm, idx_v, rows_v, sem):
        wid = lax.axis_index("s") * NC + lax.axis_index("c")
        base0 = wid * epw

        def bdy(i, carry):
            base = base0 + i * b
            pltpu.sync_copy(src_hbm.at[pl.ds(base, b)], idx_v)
            pltpu.async_copy(tsrc.at[idx_v], rows_v, sem).wait()
            pltpu.sync_copy(rows_v, os_hbm.at[pl.ds(base, b)])
            pltpu.sync_copy(dst_hbm.at[pl.ds(base, b)], idx_v)
            pltpu.async_copy(tdst.at[idx_v], rows_v, sem).wait()
            pltpu.sync_copy(rows_v, od_hbm.at[pl.ds(base, b)])
            return carry

        lax.fori_loop(0, nb, bdy, 0)

    return k


@functools.lru_cache(None)
def _deg_sc(ndst, e):
    epw = e // NW
    b = 1000
    nb = epw // b
    assert epw % b == 0
    g = (b + LANES - 1) // LANES
    buf = g * LANES
    np_ = _round_up(ndst, 256)
    span = np_ // NS

    @functools.partial(
        pl.kernel,
        mesh=_mesh(),
        compiler_params=_SC_PARAMS,
        out_type=jax.ShapeDtypeStruct((NC, np_), F32),
        scratch_types=[pltpu.VMEM((buf,), I32),
                       pltpu.VMEM((buf,), F32),
                       pltpu.VMEM((np_,), F32),
                       pltpu.VMEM((span,), F32),
                       pltpu.VMEM((span,), F32),
                       pltpu.VMEM_SHARED((NS, np_), F32)],
    )
    def k(dst_hbm, w_hbm, out_hbm, dst_v, w_v, acc_v, red_v, tmp_v, shr):
        c = lax.axis_index("c")
        s = lax.axis_index("s")
        wid = s * NC + c
        zf = jnp.zeros((LANES,), F32)
        zi = jnp.zeros((LANES,), I32)

        def z(i, carry):
            acc_v[pl.ds(i * LANES, LANES)] = zf
            return carry

        lax.fori_loop(0, np_ // LANES, z, 0)
        dst_v[pl.ds(buf - LANES, LANES)] = zi
        w_v[pl.ds(buf - LANES, LANES)] = zf

        base0 = wid * epw
        lane = lax.iota(I32, LANES)

        def bdy(i, carry):
            base = base0 + i * b
            pltpu.sync_copy(dst_hbm.at[pl.ds(base, b)], dst_v.at[pl.ds(0, b)])
            pltpu.sync_copy(w_hbm.at[pl.ds(base, b)], w_v.at[pl.ds(0, b)])

            def grp(j, cc):
                msk = (j * LANES + lane) < b
                d16 = dst_v[pl.ds(j * LANES, LANES)]
                w16 = w_v[pl.ds(j * LANES, LANES)]
                plsc.addupdate_scatter(acc_v, [d16], w16, mask=msk)
                return cc

            lax.fori_loop(0, g, grp, 0)
            return carry

        lax.fori_loop(0, nb, bdy, 0)

        pltpu.sync_copy(acc_v, shr.at[s])
        plsc.subcore_barrier()
        pltpu.sync_copy(shr.at[0, pl.ds(s * span, span)], red_v)

        def red(i, carry):
            pltpu.sync_copy(shr.at[i, pl.ds(s * span, span)], tmp_v)

            def add(j, cc):
                red_v[pl.ds(j * LANES, LANES)] = (
                    red_v[pl.ds(j * LANES, LANES)]
                    + tmp_v[pl.ds(j * LANES, LANES)])
                return cc

            lax.fori_loop(0, span // LANES, add, 0)
            return carry

        lax.fori_loop(1, NS, red, 0)
        pltpu.sync_copy(red_v, out_hbm.at[c, pl.ds(s * span, span)])

    return k


@functools.lru_cache(None)
def _agg_sc(nsrc, ndst, e):
    eps = e // NS
    b = 400
    bp = b
    nb = eps // b
    nb2 = nb // 2
    odd = nb % 2 == 1
    assert eps % b == 0 and bp % LANES == 0
    nch_n = 1
    while _round_up(-(-ndst // nch_n), 256) + 256 > 51000:
        nch_n += 1
    nch = _round_up(-(-ndst // nch_n), 256)
    np_ = nch * nch_n
    acc_rows = nch + 256
    span = nch // NS
    wch = span
    while wch > b:
        wch //= 2
    assert span % wch == 0 and wch % 8 == 0
    nwr = span // wch
    g16 = bp // LANES

    @functools.partial(
        pl.kernel,
        mesh=_mesh(),
        compiler_params=_SC_PARAMS,
        out_type=jax.ShapeDtypeStruct((NC, np_, HALF), F32),
        scratch_types=[pltpu.VMEM((bp,), I32), pltpu.VMEM((bp,), I32),
                       pltpu.VMEM((bp,), I32), pltpu.VMEM((bp,), I32),
                       pltpu.VMEM((bp,), F32), pltpu.VMEM((bp,), F32),
                       pltpu.VMEM((bp, HALF), F32),
                       pltpu.VMEM((bp, HALF), F32),
                       pltpu.VMEM_SHARED((acc_rows, HALF), F32),
                       pltpu.SemaphoreType.DMA,
                       pltpu.SemaphoreType.DMA],
    )
    def k(mcat, src_hbm, dst_hbm, w_hbm, out_hbm,
          src0, src1, dst0, dst1, w0, w1, rows0, rows1, acc, sem0, sem1):
        c = lax.axis_index("c")
        s = lax.axis_index("s")
        zf = jnp.zeros((LANES,), F32)
        srcs, dsts, ws = (src0, src1), (dst0, dst1), (w0, w1)
        rows, sems = (rows0, rows1), (sem0, sem1)
        coff = jnp.full((LANES,), 0, I32) + c * nsrc

        def load_idx(i, t):
            base = s * eps + i * b
            pltpu.sync_copy(src_hbm.at[pl.ds(base, b)], srcs[t])
            pltpu.sync_copy(dst_hbm.at[pl.ds(base, b)], dsts[t])
            pltpu.sync_copy(w_hbm.at[pl.ds(base, b)], ws[t])

            def ofs(j, cc):
                srcs[t][pl.ds(j * LANES, LANES)] = (
                    srcs[t][pl.ds(j * LANES, LANES)] + coff)
                return cc

            lax.fori_loop(0, g16, ofs, 0)

        def process(t, lo):
            pltpu.make_async_copy(mcat.at[pl.ds(0, bp)], rows[t],
                                  sems[t]).wait()
            if nch_n > 1:
                def rb(j, cc):
                    d16 = dsts[t][pl.ds(j * LANES, LANES)]
                    loc = d16 - lo
                    ok = (d16 >= lo) & (d16 < lo + nch)
                    dump = jnp.full((LANES,), nch, I32)
                    dsts[t][pl.ds(j * LANES, LANES)] = (
                        jnp.where(ok, loc, dump))
                    return cc

                lax.fori_loop(0, g16, rb, 0)

            def sc8(e0, cc):
                for u in range(8):
                    ei = e0 * 8 + u
                    wv = plsc.load_gather(
                        ws[t], [jnp.full((LANES,), 0, I32) + ei])
                    rows[t][ei, pl.ds(0, LANES)] = (
                        rows[t][ei, pl.ds(0, LANES)] * wv)
                    rows[t][ei, pl.ds(LANES, LANES)] = (
                        rows[t][ei, pl.ds(LANES, LANES)] * wv)
                return cc

            lax.fori_loop(0, bp // 8, sc8, 0)
            pltpu.sync_copy(rows[t], acc.at[dsts[t]], add=True)

        for h in range(nch_n):
            lo = h * nch

            def z(i, carry):
                rows0[i, pl.ds(0, LANES)] = zf
                rows0[i, pl.ds(LANES, LANES)] = zf
                return carry

            lax.fori_loop(0, wch, z, 0)
            for i in range(nwr):
                pltpu.sync_copy(rows0.at[pl.ds(0, wch)],
                                acc.at[pl.ds(s * span + i * wch, wch)])
            plsc.subcore_barrier()

            load_idx(0, 0)
            pltpu.async_copy(mcat.at[src0], rows0, sem0)

            def bdy(i2, carry):
                load_idx(i2 * 2 + 1, 1)
                pltpu.async_copy(mcat.at[src1], rows1, sem1)
                process(0, lo)

                if odd:
                    load_idx(i2 * 2 + 2, 0)
                    pltpu.async_copy(mcat.at[src0], rows0, sem0)
                else:
                    @pl.when(i2 < nb2 - 1)
                    def _():
                        load_idx(i2 * 2 + 2, 0)
                        pltpu.async_copy(mcat.at[src0], rows0, sem0)

                process(1, lo)
                return carry

            lax.fori_loop(0, nb2, bdy, 0)
            if odd:
                process(0, lo)

            plsc.subcore_barrier()
            for i in range(nwr):
                off = s * span + i * wch
                pltpu.sync_copy(acc.at[pl.ds(off, wch)],
                                rows0.at[pl.ds(0, wch)])
                pltpu.sync_copy(rows0.at[pl.ds(0, wch)],
                                out_hbm.at[c, pl.ds(lo + off, wch)])
            if h + 1 < nch_n:
                plsc.subcore_barrier()

    return k


def _message_pass(node, neigh, src, dst, w, deg, p):
    n = node.shape[0]
    nsrc = neigh.shape[0]
    m2 = _msg_tc(neigh, p["message"])
    mcat = m2.reshape(2 * nsrc, HALF)
    agg2 = _agg_sc(nsrc, n, src.shape[0])(mcat, src, dst, w)
    agg_lo = agg2[0, :n, :]
    agg_hi = agg2[1, :n, :]
    return _update_tc(agg_lo, agg_hi, deg, node, p)


def kernel(variable_features, constraint_features, cut_features,
           var_cons_edge_features, var_cut_edge_features,
           var_cons_edges, var_cut_edges, params):
    n_var = variable_features.shape[0]
    n_cons = constraint_features.shape[0]
    n_cut = cut_features.shape[0]
    e_vc = var_cons_edges.shape[1]
    e_vk = var_cut_edges.shape[1]

    h_var = _mlp2_tc(variable_features, params["var_emb"])
    h_cons = _mlp2_tc(constraint_features, params["cons_emb"])
    h_cut = _mlp2_tc(cut_features, params["cut_emb"])

    vc_s = var_cons_edges[0]
    vc_d = var_cons_edges[1]
    vk_s = var_cut_edges[0]
    vk_d = var_cut_edges[1]

    gs, gd = _pair_gather_sc(n_var, n_cons, e_vc)(h_var, h_cons, vc_s, vc_d)
    vc_w = _edgew_tc(gs, gd, var_cons_edge_features, params["ew_vc"])
    gs, gd = _pair_gather_sc(n_var, n_cut, e_vk)(h_var, h_cut, vk_s, vk_d)
    vk_w = _edgew_tc(gs, gd, var_cut_edge_features, params["ew_vk"])

    vc_w1 = vc_w.reshape(e_vc)
    vk_w1 = vk_w.reshape(e_vk)

    def deg_of(dstv, wv, ndst):
        parts = _deg_sc(ndst, dstv.shape[0])(dstv, wv)
        return (parts[0, :ndst] + parts[1, :ndst]).reshape(ndst, 1)

    deg_v2c = deg_of(vc_d, vc_w1, n_cons)
    deg_c2v = deg_of(vc_s, vc_w1, n_var)
    deg_v2k = deg_of(vk_d, vk_w1, n_cut)
    deg_k2v = deg_of(vk_s, vk_w1, n_var)

    for l in range(4):
        h_cons = _message_pass(h_cons, h_var, vc_s, vc_d, vc_w1, deg_v2c,
                               params["mp_v2c"][l])
        h_var = _message_pass(h_var, h_cons, vc_d, vc_s, vc_w1, deg_c2v,
                              params["mp_c2v"][l])
        h_cut = _message_pass(h_cut, h_var, vk_s, vk_d, vk_w1, deg_v2k,
                              params["mp_v2k"][l])
        h_var = _message_pass(h_var, h_cut, vk_d, vk_s, vk_w1, deg_k2v,
                              params["mp_k2v"][l])
    return h_cut
